# Initial kernel scaffold; baseline (speedup 1.0000x reference)
import jax
import jax.numpy as jnp
from jax.experimental import pallas as pl


def kernel(x, edge_index, batch, W1, b1, W2, b2, W3, b3, g0, be0, g1, be1, g2, be2, fW1, fb1, fW2, fb2):
    raise NotImplementedError("write your pallas kernel here")


# speedup vs baseline: 9.2301x; 9.2301x over previous
import functools

import jax
import jax.numpy as jnp
from jax import lax
from jax.experimental import pallas as pl
from jax.experimental.pallas import tpu as pltpu
from jax.experimental.pallas import tpu_sc as plsc

NC = 2
NS = 16
NW = NC * NS
B = 128


@functools.cache
def _make_sc_scatter(n_src, n_pad, d, e_pad):
  epw = e_pad // NW
  steps = epw // B
  rpt = n_pad // NS

  mesh = plsc.VectorSubcoreMesh(core_axis_name="c", subcore_axis_name="s")

  @functools.partial(
      pl.kernel,
      out_type=jax.ShapeDtypeStruct((NC, n_pad, d), jnp.float32),
      mesh=mesh,
      scratch_types=[
          pltpu.VMEM((B,), jnp.int32),
          pltpu.VMEM((B,), jnp.int32),
          pltpu.VMEM((B, d), jnp.float32),
          pltpu.VMEM_SHARED((n_pad, d), jnp.float32),
          pltpu.SemaphoreType.DMA,
      ],
  )
  def sc_scatter(src_hbm, row_hbm, col_hbm, zeros_hbm, out_hbm,
                 rowv, colv, rows, acc, sem):
    c = lax.axis_index("c")
    s = lax.axis_index("s")
    wid = s * NC + c
    pltpu.sync_copy(zeros_hbm.at[pl.ds(s * rpt, rpt)],
                    acc.at[pl.ds(s * rpt, rpt)])
    plsc.subcore_barrier()

    base = wid * epw

    def body(i, carry):
      off = base + i * B
      pltpu.sync_copy(row_hbm.at[pl.ds(off, B)], rowv)
      pltpu.sync_copy(col_hbm.at[pl.ds(off, B)], colv)
      pltpu.async_copy(src_hbm.at[rowv], rows, sem).wait()
      pltpu.sync_copy(rows, acc.at[colv], add=True)
      return carry

    lax.fori_loop(0, steps, body, 0)
    plsc.subcore_barrier()
    pltpu.sync_copy(acc.at[pl.ds(s * rpt, rpt)],
                    out_hbm.at[c, pl.ds(s * rpt, rpt)])

  return sc_scatter


@functools.cache
def _make_sc_deg(n_pad, e_pad):
  epw = e_pad // NW
  steps = epw // B
  mesh = plsc.VectorSubcoreMesh(core_axis_name="c", subcore_axis_name="s")

  @functools.partial(
      pl.kernel,
      out_type=jax.ShapeDtypeStruct((NW * n_pad,), jnp.float32),
      mesh=mesh,
      scratch_types=[
          pltpu.VMEM((B,), jnp.int32),
          pltpu.VMEM((n_pad,), jnp.float32),
      ],
      compiler_params=pltpu.CompilerParams(needs_layout_passes=False),
  )
  def sc_deg(col_hbm, out_hbm, colv, deg_l):
    c = lax.axis_index("c")
    s = lax.axis_index("s")
    wid = s * NC + c
    zeros = jnp.zeros((16,), jnp.float32)

    def zbody(i, carry):
      deg_l[pl.ds(i * 16, 16)] = zeros
      return carry

    lax.fori_loop(0, n_pad // 16, zbody, 0)
    ones = jnp.ones((16,), jnp.float32)
    base = wid * epw

    def body(i, carry):
      pltpu.sync_copy(col_hbm.at[pl.ds(base + i * B, B)], colv)
      for j in range(B // 16):
        idx = colv[pl.ds(j * 16, 16)]
        plsc.addupdate_scatter(deg_l, [idx], ones)
      return carry

    lax.fori_loop(0, steps, body, 0)
    pltpu.sync_copy(deg_l, out_hbm.at[pl.ds(wid * n_pad, n_pad)])

  return sc_deg


def _tc_prep_body(deg2d_ref, x_ref, w_ref, dis_ref, hn_ref):
  n = x_ref.shape[0]
  nw = deg2d_ref.shape[0]
  deg_col = lax.dot_general(
      deg2d_ref[...], jnp.ones((nw, 1), jnp.float32),
      (((0,), (0,)), ((), ())), preferred_element_type=jnp.float32)
  deg = deg_col[:n] + 1.0
  dis = lax.rsqrt(deg)
  dis_ref[...] = dis
  h = jnp.dot(x_ref[...], w_ref[...], preferred_element_type=jnp.float32)
  hn_ref[...] = h * dis


def _bn_relu(z, g, be, eps=1e-5):
  mean = jnp.mean(z, axis=0, keepdims=True)
  var = jnp.mean((z - mean) ** 2, axis=0, keepdims=True)
  zn = (z - mean) * lax.rsqrt(var + eps) * g + be
  return jnp.maximum(zn, 0.0)


def _tc_mid_body(acc_ref, hn_ref, dis_ref, b_ref, g_ref, be_ref, w_ref,
                 out_ref):
  n = hn_ref.shape[0]
  dis = dis_ref[...]
  z = dis * (acc_ref[0, :n, :] + acc_ref[1, :n, :] + hn_ref[...]) + b_ref[...]
  r = _bn_relu(z, g_ref[...], be_ref[...])
  h = jnp.dot(r, w_ref[...], preferred_element_type=jnp.float32)
  out_ref[...] = h * dis


def _tc_final_body(acc_ref, hn_ref, dis_ref, b_ref, g_ref, be_ref, batch_ref,
                   fw1_ref, fb1_ref, fw2_ref, fb2_ref, out_ref):
  n = hn_ref.shape[0]
  g_num = out_ref.shape[0]
  dis = dis_ref[...]
  z = dis * (acc_ref[0, :n, :] + acc_ref[1, :n, :] + hn_ref[...]) + b_ref[...]
  r = _bn_relu(z, g_ref[...], be_ref[...])
  gids = lax.broadcasted_iota(jnp.int32, (n, g_num), 1)
  mask = (batch_ref[...] == gids).astype(jnp.float32)
  dnum = (((0,), (0,)), ((), ()))
  sums = lax.dot_general(mask, r, dnum, preferred_element_type=jnp.float32)
  cnts = lax.dot_general(mask, jnp.ones((n, 1), jnp.float32), dnum,
                         preferred_element_type=jnp.float32)
  pooled = sums / jnp.maximum(cnts, 1.0)
  o1 = jnp.maximum(
      jnp.dot(pooled, fw1_ref[...], preferred_element_type=jnp.float32)
      + fb1_ref[...], 0.0)
  out_ref[...] = (
      jnp.dot(o1, fw2_ref[...], preferred_element_type=jnp.float32)
      + fb2_ref[...])


def kernel(x, edge_index, batch, W1, b1, W2, b2, W3, b3,
           g0, be0, g1, be1, g2, be2, fW1, fb1, fW2, fb2):
  n, d = x.shape
  e = edge_index.shape[1]
  g_num = 64
  o_dim = fW2.shape[1]

  n_pad = -(-(n + 1) // (NS * 8)) * (NS * 8)
  epw = -(-e // (NW * B)) * B
  e_pad = epw * NW

  row = edge_index[0].astype(jnp.int32)
  col = edge_index[1].astype(jnp.int32)
  pad = e_pad - e
  row_p = jnp.concatenate([row, jnp.zeros((pad,), jnp.int32)])
  col_p = jnp.concatenate([col, jnp.full((pad,), n, jnp.int32)])

  zeros_d = jnp.zeros((n_pad, d), jnp.float32)

  scd = _make_sc_scatter(n, n_pad, d, e_pad)

  deg_flat = _make_sc_deg(n_pad, e_pad)(col_p)
  deg2d = deg_flat.reshape(NW, n_pad)

  dis, hn1 = pl.pallas_call(
      _tc_prep_body,
      out_shape=(jax.ShapeDtypeStruct((n, 1), jnp.float32),
                 jax.ShapeDtypeStruct((n, d), jnp.float32)),
  )(deg2d, x, W1)

  acc1 = scd(hn1, row_p, col_p, zeros_d)

  mid = pl.pallas_call(
      _tc_mid_body,
      out_shape=jax.ShapeDtypeStruct((n, d), jnp.float32),
  )
  hn2 = mid(acc1, hn1, dis, b1, g0, be0, W2)
  acc2 = scd(hn2, row_p, col_p, zeros_d)
  hn3 = mid(acc2, hn2, dis, b2, g1, be1, W3)
  acc3 = scd(hn3, row_p, col_p, zeros_d)

  out = pl.pallas_call(
      _tc_final_body,
      out_shape=jax.ShapeDtypeStruct((g_num, o_dim), jnp.float32),
  )(acc3, hn3, dis, b3, g2, be2, batch.astype(jnp.int32).reshape(n, 1),
    fW1, fb1, fW2, fb2)
  return out
